# Initial kernel scaffold; baseline (speedup 1.0000x reference)
import functools

import jax
import jax.numpy as jnp
from jax import lax
from jax.experimental import pallas as pl
from jax.experimental.pallas import tpu as pltpu
from jax.experimental.pallas import tpu_sc as plsc

_B, _S, _D = 4, 2048, 1024
_V, _H = 128, 1024
_TOT = _S + _V
_CH = 16
_VP = _V + _CH
_NC, _NS = 2, 16
_NW = _NC * _NS
_WPB = _NW // _B
_RPW = _TOT // _WPB
_CPW = _RPW // _CH


def _mlp_body(emb_ref, w1_ref, b1_ref, w2_ref, b2_ref, out_ref):
    h = jnp.tanh(
        jnp.dot(emb_ref[...], w1_ref[...], preferred_element_type=jnp.float32)
        + b1_ref[...]
    )
    o = jnp.dot(h, w2_ref[...], preferred_element_type=jnp.float32) + b2_ref[...]
    out_ref[: _V, :] = o
    out_ref[_V:, :] = jnp.zeros((_VP - _V, _D), jnp.float32)


def _virtual_table(W_emb, W1, b1, W2, b2):
    return pl.pallas_call(
        _mlp_body,
        out_shape=jax.ShapeDtypeStruct((_VP, _D), jnp.float32),
    )(W_emb, W1, b1.reshape(1, _H), W2, b2.reshape(1, _D))


_sc_mesh = plsc.VectorSubcoreMesh(core_axis_name="c", subcore_axis_name="s")


@functools.partial(
    pl.kernel,
    mesh=_sc_mesh,
    out_type=jax.ShapeDtypeStruct((_B, _TOT, _D), jnp.float32),
    scratch_types=[
        pltpu.VMEM((_CH, _D), jnp.float32),
        pltpu.VMEM((_CH, _D), jnp.float32),
        pltpu.SMEM((16,), jnp.int32),
    ],
)
def _sc_assemble(inputs_hbm, vpad_hbm, seq_hbm, out_hbm, buf, zbuf, lens):
    wid = lax.axis_index("s") * _NC + lax.axis_index("c")
    b = wid // _WPB
    t0 = (wid % _WPB) * _RPW

    pltpu.sync_copy(seq_hbm, lens)
    pltpu.sync_copy(vpad_hbm.at[pl.ds(_V, _CH)], zbuf)
    ln = lens[b]

    def chunk(c, carry):
        r = t0 + c * _CH
        off = r - ln

        @pl.when(off <= -_CH)
        def _():
            pltpu.sync_copy(inputs_hbm.at[b, pl.ds(r, _CH)], buf)
            pltpu.sync_copy(buf, out_hbm.at[b, pl.ds(r, _CH)])

        @pl.when((off > -_CH) & (off < 0))
        def _():
            def row(j, cr):
                t = r + j

                def from_inputs(_):
                    pltpu.sync_copy(
                        inputs_hbm.at[b, pl.ds(t, 1)], buf.at[pl.ds(j, 1)]
                    )
                    return 0

                def from_virtual(_):
                    pltpu.sync_copy(
                        vpad_hbm.at[pl.ds(t - ln, 1)], buf.at[pl.ds(j, 1)]
                    )
                    return 0

                lax.cond(t < ln, from_inputs, from_virtual, 0)
                return cr

            lax.fori_loop(0, _CH, row, 0)
            pltpu.sync_copy(buf, out_hbm.at[b, pl.ds(r, _CH)])

        @pl.when((off >= 0) & (off <= _V))
        def _():
            pltpu.sync_copy(vpad_hbm.at[pl.ds(off, _CH)], buf)
            pltpu.sync_copy(buf, out_hbm.at[b, pl.ds(r, _CH)])

        @pl.when(off > _V)
        def _():
            pltpu.sync_copy(zbuf, out_hbm.at[b, pl.ds(r, _CH)])

        return carry

    lax.fori_loop(0, _CPW, chunk, 0)


def kernel(inputs, seq_len, W_emb, W1, b1, W2, b2):
    vpad = _virtual_table(W_emb, W1, b1, W2, b2)
    seq_pad = jnp.zeros((16,), jnp.int32).at[: _B].set(seq_len.astype(jnp.int32))
    out = _sc_assemble(inputs, vpad, seq_pad)
    return out, seq_len + _V


# speedup vs baseline: 4.4233x; 4.4233x over previous
import functools

import jax
import jax.numpy as jnp
from jax import lax
from jax.experimental import pallas as pl
from jax.experimental.pallas import tpu as pltpu
from jax.experimental.pallas import tpu_sc as plsc

_B, _S, _D = 4, 2048, 1024
_V, _H = 128, 1024
_TOT = _S + _V
_CH = 16
_VP = _V + _CH
_NC, _NS = 2, 16
_NW = _NC * _NS
_WPB = _NW // _B
_RPW = _TOT // _WPB
_CPW = _RPW // _CH


def _mlp_body(emb_ref, w1_ref, b1_ref, w2_ref, b2_ref, out_ref):
    h = jnp.tanh(
        jnp.dot(emb_ref[...], w1_ref[...], preferred_element_type=jnp.float32)
        + b1_ref[...]
    )
    o = jnp.dot(h, w2_ref[...], preferred_element_type=jnp.float32) + b2_ref[...]
    out_ref[: _V, :] = o
    out_ref[_V:, :] = jnp.zeros((_VP - _V, _D), jnp.float32)


def _virtual_table(W_emb, W1, b1, W2, b2):
    return pl.pallas_call(
        _mlp_body,
        out_shape=jax.ShapeDtypeStruct((_VP, _D), jnp.float32),
    )(W_emb, W1, b1.reshape(1, _H), W2, b2.reshape(1, _D))


_sc_mesh = plsc.VectorSubcoreMesh(core_axis_name="c", subcore_axis_name="s")


@functools.partial(
    pl.kernel,
    mesh=_sc_mesh,
    out_type=jax.ShapeDtypeStruct((_B, _TOT, _D), jnp.float32),
    scratch_types=[
        pltpu.VMEM((_CH, _D), jnp.float32),
        pltpu.VMEM((_CH, _D), jnp.float32),
        pltpu.VMEM((16,), jnp.int32),
    ],
    compiler_params=pltpu.CompilerParams(
        use_tc_tiling_on_sc=False, needs_layout_passes=False
    ),
)
def _sc_assemble(inputs_hbm, vpad_hbm, seq_hbm, out_hbm, buf, zbuf, seqv):
    wid = lax.axis_index("s") * _NC + lax.axis_index("c")
    b = wid // _WPB
    t0 = (wid % _WPB) * _RPW

    pltpu.sync_copy(seq_hbm, seqv)
    pltpu.sync_copy(vpad_hbm.at[pl.ds(_V, _CH)], zbuf)
    lane = lax.iota(jnp.int32, 16)
    ln = jnp.max(jnp.where(lane == b, seqv[...], 0))

    def chunk(c, carry):
        r = t0 + c * _CH
        off = r - ln

        @pl.when(off <= -_CH)
        def _():
            pltpu.sync_copy(inputs_hbm.at[b, pl.ds(r, _CH)], buf)
            pltpu.sync_copy(buf, out_hbm.at[b, pl.ds(r, _CH)])

        @pl.when((off > -_CH) & (off < 0))
        def _():
            def row(j, cr):
                t = r + j

                def from_inputs(_):
                    pltpu.sync_copy(
                        inputs_hbm.at[b, pl.ds(t, 1)], buf.at[pl.ds(j, 1)]
                    )
                    return 0

                def from_virtual(_):
                    pltpu.sync_copy(
                        vpad_hbm.at[pl.ds(t - ln, 1)], buf.at[pl.ds(j, 1)]
                    )
                    return 0

                lax.cond(t < ln, from_inputs, from_virtual, 0)
                return cr

            lax.fori_loop(0, _CH, row, 0)
            pltpu.sync_copy(buf, out_hbm.at[b, pl.ds(r, _CH)])

        @pl.when((off >= 0) & (off <= _V))
        def _():
            pltpu.sync_copy(vpad_hbm.at[pl.ds(off, _CH)], buf)
            pltpu.sync_copy(buf, out_hbm.at[b, pl.ds(r, _CH)])

        @pl.when(off > _V)
        def _():
            pltpu.sync_copy(zbuf, out_hbm.at[b, pl.ds(r, _CH)])

        return carry

    lax.fori_loop(0, _CPW, chunk, 0)


def kernel(inputs, seq_len, W_emb, W1, b1, W2, b2):
    vpad = _virtual_table(W_emb, W1, b1, W2, b2)
    seq_pad = jnp.zeros((16,), jnp.int32).at[: _B].set(seq_len.astype(jnp.int32))
    out = _sc_assemble(inputs, vpad, seq_pad)
    return out, seq_len + _V
